# baseline (speedup 1.0000x reference)
import functools

import jax
import jax.numpy as jnp
from jax import lax
from jax.experimental import pallas as pl
from jax.experimental.pallas import tpu as pltpu
from jax.experimental.pallas import tpu_sc as plsc

_NC = 2
_NS = 16
_NW = _NC * _NS
_D = 64
_L = 16
_G = 128
_KG = 8
_C = _KG * _G
_SCALE = 8.0


@functools.lru_cache(maxsize=None)
def _build(B):
    W = B // _NW
    n_chunks = W // _C
    mesh = plsc.VectorSubcoreMesh(
        core_axis_name="c", subcore_axis_name="s",
        num_cores=_NC, num_subcores=_NS)

    @functools.partial(
        pl.kernel,
        out_type=jax.ShapeDtypeStruct((B, _D), jnp.float32),
        mesh=mesh,
        scratch_types=[
            pltpu.VMEM((_KG, _G), jnp.int32),
            pltpu.VMEM((_C, _D), jnp.float32),
            pltpu.SemaphoreType.DMA,
        ],
        compiler_params=pltpu.CompilerParams(use_tc_tiling_on_sc=False),
    )
    def k(tok_hbm, table_hbm, out_hbm, idx_v, rows_v, gsem):
        wid = lax.axis_index("s") * _NC + lax.axis_index("c")
        chunk0 = wid * n_chunks

        @pl.loop(0, n_chunks)
        def _chunk(kc):
            pltpu.sync_copy(tok_hbm.at[chunk0 + kc], idx_v)
            descs = [
                pltpu.async_copy(table_hbm.at[idx_v.at[j]],
                                 rows_v.at[pl.ds(j * _G, _G)], gsem)
                for j in range(_KG)
            ]
            for d in descs:
                d.wait()

            @pl.loop(0, _C)
            def _scale(r):
                for c in range(_D // _L):
                    sl = pl.ds(c * _L, _L)
                    rows_v[r, sl] = rows_v[r, sl] * _SCALE

            pltpu.sync_copy(rows_v,
                            out_hbm.at[pl.ds(wid * W + kc * _C, _C)])

    return k


def kernel(tokens, embedding_weight):
    n, s = tokens.shape
    B = n * s
    tok = tokens.reshape(B // _C, _KG, _G).astype(jnp.int32)
    out = _build(B)(tok, embedding_weight)
    return out.reshape(n, s, _D)


# speedup vs baseline: 1.1098x; 1.1098x over previous
import functools

import jax
import jax.numpy as jnp
from jax import lax
from jax.experimental import pallas as pl
from jax.experimental.pallas import tpu as pltpu
from jax.experimental.pallas import tpu_sc as plsc

_NC = 2
_NS = 16
_NW = _NC * _NS
_D = 64
_L = 16
_G = 128
_KG = 5
_C = _KG * _G
_SCALE = 8.0


@functools.lru_cache(maxsize=None)
def _build(B):
    W = B // _NW
    n_chunks = W // _C
    idx_rows = W // _G
    mesh = plsc.VectorSubcoreMesh(
        core_axis_name="c", subcore_axis_name="s",
        num_cores=_NC, num_subcores=_NS)

    @functools.partial(
        pl.kernel,
        out_type=jax.ShapeDtypeStruct((B, _D), jnp.float32),
        mesh=mesh,
        scratch_types=[
            pltpu.VMEM((idx_rows, _G), jnp.int32),
            pltpu.VMEM((_C, _D), jnp.float32),
            pltpu.VMEM((_C, _D), jnp.float32),
            pltpu.SemaphoreType.DMA,
            pltpu.SemaphoreType.DMA,
            pltpu.SemaphoreType.DMA,
            pltpu.SemaphoreType.DMA,
        ],
        compiler_params=pltpu.CompilerParams(use_tc_tiling_on_sc=False),
    )
    def k(tok_hbm, table_hbm, out_hbm, idx_v, rows0, rows1, g0, g1, s0, s1):
        wid = lax.axis_index("s") * _NC + lax.axis_index("c")
        base = wid * W
        rows = (rows0, rows1)
        gsems = (g0, g1)
        ssems = (s0, s1)

        pltpu.sync_copy(tok_hbm.at[pl.ds(wid * idx_rows, idx_rows)], idx_v)

        def fire_gather(kc, b):
            for j in range(_KG):
                pltpu.async_copy(table_hbm.at[idx_v.at[kc * _KG + j]],
                                 rows[b].at[pl.ds(j * _G, _G)], gsems[b])

        def drain_gather(b):
            pltpu.make_async_copy(table_hbm.at[pl.ds(0, _C)],
                                  rows[b], gsems[b]).wait()

        def fire_scatter(kc, b):
            pltpu.async_copy(rows[b],
                             out_hbm.at[pl.ds(base + kc * _C, _C)], ssems[b])

        def drain_scatter(b):
            pltpu.make_async_copy(rows[b],
                                  out_hbm.at[pl.ds(base, _C)],
                                  ssems[b]).wait()

        def scale(b):
            @plsc.parallel_loop(0, _C, unroll=8)
            def _scale(r):
                for c in range(_D // _L):
                    sl = pl.ds(c * _L, _L)
                    rows[b][r, sl] = rows[b][r, sl] * _SCALE

        fire_gather(0, 0)

        @pl.loop(0, n_chunks, step=2)
        def _chunk(k0):
            for b in (0, 1):
                kc = k0 + b

                @pl.when(kc + 1 < n_chunks)
                def _prefetch():
                    @pl.when(kc >= 1)
                    def _reuse():
                        drain_scatter(1 - b)
                    fire_gather(kc + 1, 1 - b)

                drain_gather(b)
                scale(b)
                fire_scatter(kc, b)

        drain_scatter(0)
        drain_scatter(1)

    return k


def kernel(tokens, embedding_weight):
    n, s = tokens.shape
    B = n * s
    tok = tokens.reshape(B // _G, _G).astype(jnp.int32)
    out = _build(B)(tok, embedding_weight)
    return out.reshape(n, s, _D)
